# baseline (speedup 1.0000x reference)
import jax
import jax.numpy as jnp
from jax.experimental import pallas as pl
from jax.experimental.pallas import tpu as pltpu

N_CLASS = 21
N_FG = N_CLASS - 1
N_ROI = 1000
N_PAD = 1024
IMG_H, IMG_W = 600, 800
SCORE_LOW = 0.05
NMS_THRESH = 0.3
R_SCALE = NMS_THRESH / (1.0 + NMS_THRESH)
TILE = 128
NEG = -1e30
CPB = 5


def _tile_any(cols, rows, i0, mode, jlt):
    by1_c, bx1_c, by2_c, bx2_c, ra_c, s_c = cols
    by1_r, bx1_r, by2_r, bx2_r, ra_r, s_r = (
        r[:, i0 : i0 + TILE] for r in rows
    )
    iy1 = jnp.maximum(by1_c, by1_r)
    ix1 = jnp.maximum(bx1_c, bx1_r)
    iy2 = jnp.minimum(by2_c, by2_r)
    ix2 = jnp.minimum(bx2_c, bx2_r)
    inter = jnp.maximum(iy2 - iy1, 0.0) * jnp.maximum(ix2 - ix1, 0.0)
    over = inter > (ra_c + ra_r)
    if mode == "gt":
        higher = s_c > s_r
    elif mode == "ge":
        higher = s_c >= s_r
    else:
        higher = (s_c > s_r) | ((s_c == s_r) & jlt)
    return jnp.any(over & higher, axis=0, keepdims=True)


def _nms_kernel(
    rois_ref, loc_ref, score_ref, bbox_out, score_out, locT_s, st_s, rt_s
):
    p = pl.program_id(0)

    @pl.when(p == 0)
    def _():
        locT_s[...] = jnp.full((N_CLASS, 4, N_PAD), NEG, jnp.float32)
        locT_s[:, :, :N_ROI] = loc_ref[...].T.reshape(N_CLASS, 4, N_ROI)
        st_s[...] = jnp.full((24, N_PAD), NEG, jnp.float32)
        st_s[0:N_CLASS, :N_ROI] = score_ref[...].T
        rt_s[...] = jnp.full((8, N_PAD), NEG, jnp.float32)
        rt_s[0:4, :N_ROI] = rois_ref[...].T

        sc = st_s[...]
        rm = jnp.max(sc, axis=0, keepdims=True)
        es = jnp.exp(sc - rm)
        rs = jnp.sum(es, axis=0, keepdims=True)
        st_s[...] = es / rs

    roisT = rt_s[...]
    sy1, sx1, sy2, sx2 = (roisT[k : k + 1, :] for k in range(4))
    src_h = sy2 - sy1
    src_w = sx2 - sx1
    src_cy = sy1 + 0.5 * src_h
    src_cx = sx1 + 0.5 * src_w

    jlt = jax.lax.broadcasted_iota(
        jnp.int32, (TILE, 1), 0
    ) < jax.lax.broadcasted_iota(jnp.int32, (1, TILE), 1)

    def body(i, _):
        cls = p * CPB + i + 1
        ld = locT_s[cls]
        dy, dx, dh, dw = (ld[k : k + 1, :] for k in range(4))
        prob_r = st_s[pl.ds(cls, 1), :]
        s_r = jnp.where(prob_r > SCORE_LOW, prob_r, 0.0)

        cy = dy * src_h + src_cy
        cx = dx * src_w + src_cx
        h = jnp.exp(dh) * src_h
        w = jnp.exp(dw) * src_w
        by1_r = jnp.clip(cy - 0.5 * h, 0.0, float(IMG_H))
        bx1_r = jnp.clip(cx - 0.5 * w, 0.0, float(IMG_W))
        by2_r = jnp.clip(cy + 0.5 * h, 0.0, float(IMG_H))
        bx2_r = jnp.clip(cx + 0.5 * w, 0.0, float(IMG_W))
        ra_r = R_SCALE * (
            jnp.maximum(by2_r - by1_r, 0.0) * jnp.maximum(bx2_r - bx1_r, 0.0)
        )
        rows = (by1_r, bx1_r, by2_r, bx2_r, ra_r, s_r)

        pack = jnp.concatenate(
            [by1_r, bx1_r, by2_r, bx2_r, ra_r, s_r, ra_r, s_r], axis=0
        )
        packT = pack.T

        supp = [None] * (N_PAD // TILE)
        for j0 in range(0, N_PAD, TILE):
            tp = packT[j0 : j0 + TILE, :]
            cols = tuple(tp[:, k : k + 1] for k in range(6))
            for it, i0 in enumerate(range(0, N_PAD, TILE)):
                if j0 == i0:
                    mode = "band"
                elif j0 < i0:
                    mode = "ge"
                else:
                    mode = "gt"
                t = _tile_any(cols, rows, i0, mode, jlt)
                supp[it] = t if supp[it] is None else (supp[it] | t)
        suppressed = jnp.concatenate(supp, axis=1)

        keep = jnp.logical_not(suppressed) & (s_r > SCORE_LOW)
        out_s = jnp.where(keep, s_r, 0.0)

        bbox_out[pl.ds(i, 1)] = packT[:N_ROI, 0:4].reshape(1, N_ROI, 4)
        score_out[pl.ds(i, 1)] = out_s[:, :N_ROI].reshape(1, 1, N_ROI)
        return 0

    jax.lax.fori_loop(0, CPB, body, 0)


@jax.jit
def kernel(rois, roi_cls_loc, roi_score):
    f = jnp.float32
    bboxes, scores = pl.pallas_call(
        _nms_kernel,
        grid=(N_FG // CPB,),
        in_specs=[
            pl.BlockSpec((N_ROI, 4), lambda c: (0, 0)),
            pl.BlockSpec((N_ROI, 4 * N_CLASS), lambda c: (0, 0)),
            pl.BlockSpec((N_ROI, N_CLASS), lambda c: (0, 0)),
        ],
        out_specs=[
            pl.BlockSpec((CPB, N_ROI, 4), lambda c: (c, 0, 0)),
            pl.BlockSpec((CPB, 1, N_ROI), lambda c: (c, 0, 0)),
        ],
        out_shape=[
            jax.ShapeDtypeStruct((N_FG, N_ROI, 4), f),
            jax.ShapeDtypeStruct((N_FG, 1, N_ROI), f),
        ],
        scratch_shapes=[
            pltpu.VMEM((N_CLASS, 4, N_PAD), f),
            pltpu.VMEM((24, N_PAD), f),
            pltpu.VMEM((8, N_PAD), f),
        ],
    )(rois, roi_cls_loc, roi_score)

    return bboxes, scores[:, 0, :]


# speedup vs baseline: 1.0024x; 1.0024x over previous
import jax
import jax.numpy as jnp
from jax.experimental import pallas as pl
from jax.experimental.pallas import tpu as pltpu

N_CLASS = 21
N_FG = N_CLASS - 1
N_ROI = 1000
N_PAD = 1024
IMG_H, IMG_W = 600, 800
SCORE_LOW = 0.05
NMS_THRESH = 0.3
R_SCALE = NMS_THRESH / (1.0 + NMS_THRESH)
TILE = 128
NEG = -1e30


def _tile_any(cols, rows, i0, mode, jlt):
    by1_c, bx1_c, by2_c, bx2_c, ra_c, s_c = cols
    by1_r, bx1_r, by2_r, bx2_r, ra_r, s_r = (
        r[:, i0 : i0 + TILE] for r in rows
    )
    iy1 = jnp.maximum(by1_c, by1_r)
    ix1 = jnp.maximum(bx1_c, bx1_r)
    iy2 = jnp.minimum(by2_c, by2_r)
    ix2 = jnp.minimum(bx2_c, bx2_r)
    inter = jnp.maximum(iy2 - iy1, 0.0) * jnp.maximum(ix2 - ix1, 0.0)
    over = inter > (ra_c + ra_r)
    if mode == "gt":
        higher = s_c > s_r
    elif mode == "ge":
        higher = s_c >= s_r
    else:
        higher = (s_c > s_r) | ((s_c == s_r) & jlt)
    return jnp.any(over & higher, axis=0, keepdims=True)


def _nms_kernel(
    rois_ref, loc_ref, score_ref, bbox_out, score_out, locT_s, st_s, rt_s
):
    locT_s[...] = jnp.full((N_CLASS, 4, N_PAD), NEG, jnp.float32)
    locT_s[:, :, :N_ROI] = loc_ref[...].T.reshape(N_CLASS, 4, N_ROI)
    st_s[...] = jnp.full((24, N_PAD), NEG, jnp.float32)
    st_s[0:N_CLASS, :N_ROI] = score_ref[...].T
    rt_s[...] = jnp.full((8, N_PAD), NEG, jnp.float32)
    rt_s[0:4, :N_ROI] = rois_ref[...].T

    sc = st_s[...]
    rm = jnp.max(sc, axis=0, keepdims=True)
    es = jnp.exp(sc - rm)
    rs = jnp.sum(es, axis=0, keepdims=True)
    prob = es / rs
    st_s[...] = prob

    roisT = rt_s[...]
    sy1, sx1, sy2, sx2 = (roisT[k : k + 1, :] for k in range(4))
    src_h = sy2 - sy1
    src_w = sx2 - sx1
    src_cy = sy1 + 0.5 * src_h
    src_cx = sx1 + 0.5 * src_w

    jlt = jax.lax.broadcasted_iota(
        jnp.int32, (TILE, 1), 0
    ) < jax.lax.broadcasted_iota(jnp.int32, (1, TILE), 1)

    def body(i, _):
        cls = i + 1
        ld = locT_s[cls]
        dy, dx, dh, dw = (ld[k : k + 1, :] for k in range(4))
        prob_r = st_s[pl.ds(cls, 1), :]
        s_r = jnp.where(prob_r > SCORE_LOW, prob_r, 0.0)

        cy = dy * src_h + src_cy
        cx = dx * src_w + src_cx
        h = jnp.exp(dh) * src_h
        w = jnp.exp(dw) * src_w
        by1_r = jnp.clip(cy - 0.5 * h, 0.0, float(IMG_H))
        bx1_r = jnp.clip(cx - 0.5 * w, 0.0, float(IMG_W))
        by2_r = jnp.clip(cy + 0.5 * h, 0.0, float(IMG_H))
        bx2_r = jnp.clip(cx + 0.5 * w, 0.0, float(IMG_W))
        ra_r = R_SCALE * (
            jnp.maximum(by2_r - by1_r, 0.0) * jnp.maximum(bx2_r - bx1_r, 0.0)
        )
        rows = (by1_r, bx1_r, by2_r, bx2_r, ra_r, s_r)

        pack = jnp.concatenate(
            [by1_r, bx1_r, by2_r, bx2_r, ra_r, s_r, ra_r, s_r], axis=0
        )
        packT = pack.T

        supp = [None] * (N_PAD // TILE)
        for j0 in range(0, N_PAD, TILE):
            tp = packT[j0 : j0 + TILE, :]
            cols = tuple(tp[:, k : k + 1] for k in range(6))
            for it, i0 in enumerate(range(0, N_PAD, TILE)):
                if j0 == i0:
                    mode = "band"
                elif j0 < i0:
                    mode = "ge"
                else:
                    mode = "gt"
                t = _tile_any(cols, rows, i0, mode, jlt)
                supp[it] = t if supp[it] is None else (supp[it] | t)
        suppressed = jnp.concatenate(supp, axis=1)

        keep = jnp.logical_not(suppressed) & (s_r > SCORE_LOW)
        out_s = jnp.where(keep, s_r, 0.0)

        bbox_out[pl.ds(i, 1)] = packT[:N_ROI, 0:4].reshape(1, N_ROI, 4)
        score_out[pl.ds(i, 1)] = out_s[:, :N_ROI].reshape(1, 1, N_ROI)
        return 0

    jax.lax.fori_loop(0, N_FG, body, 0)


@jax.jit
def kernel(rois, roi_cls_loc, roi_score):
    f = jnp.float32
    bboxes, scores = pl.pallas_call(
        _nms_kernel,
        grid=(1,),
        in_specs=[
            pl.BlockSpec((N_ROI, 4), lambda c: (0, 0)),
            pl.BlockSpec((N_ROI, 4 * N_CLASS), lambda c: (0, 0)),
            pl.BlockSpec((N_ROI, N_CLASS), lambda c: (0, 0)),
        ],
        out_specs=[
            pl.BlockSpec((N_FG, N_ROI, 4), lambda c: (0, 0, 0)),
            pl.BlockSpec((N_FG, 1, N_ROI), lambda c: (0, 0, 0)),
        ],
        out_shape=[
            jax.ShapeDtypeStruct((N_FG, N_ROI, 4), f),
            jax.ShapeDtypeStruct((N_FG, 1, N_ROI), f),
        ],
        scratch_shapes=[
            pltpu.VMEM((N_CLASS, 4, N_PAD), f),
            pltpu.VMEM((24, N_PAD), f),
            pltpu.VMEM((8, N_PAD), f),
        ],
    )(rois, roi_cls_loc, roi_score)

    return bboxes, scores[:, 0, :]
